# baseline (speedup 1.0000x reference)
import jax
import jax.numpy as jnp
from jax.experimental import pallas as pl
from jax.experimental.pallas import tpu as pltpu

_TEMP = 0.05
_NB = 4096


def _logits_t_body(xt_ref, ft_ref, o_ref):
    xt = xt_ref[...]
    norm = jnp.sqrt(jnp.sum(xt * xt, axis=0, keepdims=True))
    xs = xt * ((1.0 / _TEMP) / jnp.maximum(norm, 1e-12))
    o_ref[...] = jax.lax.dot_general(
        ft_ref[...],
        xs,
        (((0,), (0,)), ((), ())),
        preferred_element_type=jnp.float32,
    )


def _logits_t(xt, ft):
    d, b = xt.shape
    n = ft.shape[1]
    return pl.pallas_call(
        _logits_t_body,
        grid=(pl.cdiv(n, _NB),),
        in_specs=[
            pl.BlockSpec((d, b), lambda i: (0, 0)),
            pl.BlockSpec((d, _NB), lambda i: (0, i)),
        ],
        out_specs=pl.BlockSpec((_NB, b), lambda i: (i, 0)),
        out_shape=jax.ShapeDtypeStruct((n, b), jnp.float32),
        compiler_params=pltpu.CompilerParams(
            dimension_semantics=("arbitrary",),
        ),
    )(xt, ft)


def kernel(inputs, targets, features):
    del targets
    xt = jnp.transpose(inputs)
    ft = jnp.transpose(features)
    out_t = _logits_t(xt, ft)
    return jnp.transpose(out_t)
